# Initial kernel scaffold; baseline (speedup 1.0000x reference)
import jax
import jax.numpy as jnp
from jax.experimental import pallas as pl


def kernel(x, tables, W):
    raise NotImplementedError("write your pallas kernel here")


# speedup vs baseline: 1.1832x; 1.1832x over previous
import functools

import numpy as np
import jax
import jax.numpy as jnp
from jax import lax
from jax.experimental import pallas as pl
from jax.experimental.pallas import tpu as pltpu
from jax.experimental.pallas import tpu_sc as plsc

F = 26
B = 26
V = 100000
D = 32
K = 4
P = B * (B - 1) // 2
L = 16
NW = 32

NROW = B * F
NROW_PAD = 688
PPAD = 336
PMAX = 11
OW = PMAX * D


def _body(xf_hbm, tab_hbm, wf_hbm, ridx_hbm, cidx_hbm, out_hbm,
          x_v, idx_v, e_v, w_v, r_v, c_v, obuf, sem):
    wid = lax.axis_index("s") * 2 + lax.axis_index("c")
    lanes = lax.iota(jnp.int32, L)

    pltpu.sync_copy(xf_hbm, x_v)
    pltpu.sync_copy(wf_hbm, w_v)
    pltpu.sync_copy(ridx_hbm, r_v)
    pltpu.sync_copy(cidx_hbm, c_v)

    for i in range(NROW_PAD // L):
        pos = lanes + (i * L)
        fld = lax.rem(pos, F)
        idx_v[pl.ds(i * L, L)] = x_v[pl.ds(i * L, L)] + fld * V

    copies = []
    off = 0
    for n in (128, 128, 128, 128, 128, 48):
        copies.append(
            pltpu.async_copy(tab_hbm.at[idx_v.at[pl.ds(off, n)]],
                             e_v.at[pl.ds(off, n)], sem))
        off += n
    for cp in copies:
        cp.wait()

    p0 = wid * 10 + jnp.minimum(wid, 5)
    npair = jnp.where(wid < 5, PMAX, PMAX - 1)
    pvec = p0 + lanes
    mask = lanes < npair

    r_vec = plsc.load_gather(r_v, [pvec])
    c_vec = plsc.load_gather(c_v, [pvec])

    score = None
    for k in range(K):
        wr = plsc.load_gather(w_v, [r_vec * K + k])
        wc = plsc.load_gather(w_v, [c_vec * K + k])
        prod = wr * wc
        score = prod if score is None else score + prod

    rrow0 = r_vec * F
    crow0 = c_vec * F
    ocol = lanes * D

    def fbody(f, carry):
        rrow = rrow0 + f
        crow = crow0 + f
        frep = jnp.zeros((L,), jnp.int32) + f
        for d in range(D):
            dv = jnp.full((L,), d, jnp.int32)
            er = plsc.load_gather(e_v, [rrow, dv])
            ec = plsc.load_gather(e_v, [crow, dv])
            plsc.store_scatter(obuf, [frep, ocol + d], score * er * ec,
                               mask=mask)
        return carry

    lax.fori_loop(0, F, fbody, 0)

    base = p0 * D
    out_copies = [
        pltpu.async_copy(obuf.at[f, pl.ds(0, (PMAX - 1) * D)],
                         out_hbm.at[f, pl.ds(base, (PMAX - 1) * D)], sem)
        for f in range(F)
    ]
    for cp in out_copies:
        cp.wait()

    @pl.when(wid < 5)
    def _():
        tail = [
            pltpu.async_copy(obuf.at[f, pl.ds((PMAX - 1) * D, D)],
                             out_hbm.at[f, pl.ds(base + (PMAX - 1) * D, D)],
                             sem)
            for f in range(F)
        ]
        for cp in tail:
            cp.wait()


@jax.jit
def _ffm_sc(x_flat, tab_flat, w_flat, ridx, cidx):
    mesh = plsc.VectorSubcoreMesh(core_axis_name="c", subcore_axis_name="s")
    return pl.kernel(
        _body,
        out_type=jax.ShapeDtypeStruct((F, P * D), jnp.float32),
        mesh=mesh,
        compiler_params=pltpu.CompilerParams(use_tc_tiling_on_sc=False,
                                             needs_layout_passes=False),
        scratch_types=[
            pltpu.VMEM((NROW_PAD,), jnp.int32),
            pltpu.VMEM((NROW_PAD,), jnp.int32),
            pltpu.VMEM((NROW_PAD, D), jnp.float32),
            pltpu.VMEM((F * K,), jnp.float32),
            pltpu.VMEM((PPAD,), jnp.int32),
            pltpu.VMEM((PPAD,), jnp.int32),
            pltpu.VMEM((F, OW), jnp.float32),
            pltpu.SemaphoreType.DMA,
        ],
    )(x_flat, tab_flat, w_flat, ridx, cidx)


def kernel(x, tables, W):
    x_flat = jnp.pad(x.reshape(-1).astype(jnp.int32), (0, NROW_PAD - NROW))
    tab_flat = tables.reshape(F * V, D)
    w_flat = W.reshape(-1)
    r_np, c_np = np.triu_indices(B, k=1)
    ridx = jnp.pad(jnp.asarray(r_np, jnp.int32), (0, PPAD - P))
    cidx = jnp.pad(jnp.asarray(c_np, jnp.int32), (0, PPAD - P))
    return _ffm_sc(x_flat, tab_flat, w_flat, ridx, cidx)


# speedup vs baseline: 4.3022x; 3.6362x over previous
import numpy as np
import jax
import jax.numpy as jnp
from jax import lax
from jax.experimental import pallas as pl
from jax.experimental.pallas import tpu as pltpu
from jax.experimental.pallas import tpu_sc as plsc

F = 26
B = 26
V = 100000
D = 32
K = 4
P = B * (B - 1) // 2
L = 16

BPAD = 32
PPAD = 336
NCHUNK_FULL = P // L
TAIL = P - NCHUNK_FULL * L
NBUF = 4


def _body(xt_hbm, tab_hbm, wf_hbm, ridx_hbm, cidx_hbm, out_hbm,
          xt_v, w_v, r_v, c_v, e_v, obuf, stg, sems, osem):
    fid = lax.axis_index("s") * 2 + lax.axis_index("c")
    lanes = lax.iota(jnp.int32, L)

    pltpu.sync_copy(xt_hbm, xt_v)
    pltpu.sync_copy(wf_hbm, w_v)
    pltpu.sync_copy(ridx_hbm, r_v)
    pltpu.sync_copy(cidx_hbm, c_v)

    @pl.when(fid < F)
    def _():
        xbase = pl.multiple_of(fid * BPAD, BPAD)
        g_lo = xt_v[pl.ds(xbase, L)] + fid * V
        g_hi = xt_v[pl.ds(xbase + L, L)] + fid * V

        descs = [None] * NBUF
        gs = [None] * B

        def extract(b):
            g = gs[b]
            rm = g - (g // 8) * 8
            rsp = jnp.zeros((L,), jnp.int32) + rm
            s = stg[b % NBUF]
            lo = plsc.load_gather(s, [rsp, lanes])
            hi = plsc.load_gather(s, [rsp, lanes + L])
            e_v[pl.ds(b * D, L)] = lo
            e_v[pl.ds(b * D + L, L)] = hi

        for b in range(B):
            g = g_lo[b] if b < L else g_hi[b - L]
            gs[b] = g
            descs[b % NBUF] = pltpu.async_copy(
                tab_hbm.at[g // 8], stg[b % NBUF], sems[b % NBUF])
            if b >= NBUF - 1:
                descs[(b - NBUF + 1) % NBUF].wait()
                extract(b - NBUF + 1)
        for b in range(B - NBUF + 1, B):
            descs[b % NBUF].wait()
            extract(b)

        iota_h = [lanes, lanes + L]

        def do_pair(p_off, r, c, s):
            rsp = jnp.zeros((L,), jnp.int32) + r * D
            csp = jnp.zeros((L,), jnp.int32) + c * D
            sv = jnp.zeros((L,), jnp.float32) + s
            for h in range(2):
                er = plsc.load_gather(e_v, [rsp + iota_h[h]])
                ec = plsc.load_gather(e_v, [csp + iota_h[h]])
                obuf[pl.ds(p_off + h * L, L)] = er * ec * sv

        def chunk_scores(r_vec, c_vec):
            score = None
            for k in range(K):
                wr = plsc.load_gather(w_v, [r_vec * K + k])
                wc = plsc.load_gather(w_v, [c_vec * K + k])
                prod = wr * wc
                score = prod if score is None else score + prod
            return score

        def chunk_body(ch, carry):
            cbase = pl.multiple_of(ch * L, L)
            r_vec = r_v[pl.ds(cbase, L)]
            c_vec = c_v[pl.ds(cbase, L)]
            s_vec = chunk_scores(r_vec, c_vec)
            pbase = pl.multiple_of(ch * (L * D), L * D)
            for j in range(L):
                do_pair(pbase + j * D, r_vec[j], c_vec[j], s_vec[j])
            return carry

        lax.fori_loop(0, NCHUNK_FULL, chunk_body, 0)

        r_vec = r_v[pl.ds(NCHUNK_FULL * L, L)]
        c_vec = c_v[pl.ds(NCHUNK_FULL * L, L)]
        s_vec = chunk_scores(r_vec, c_vec)
        for j in range(TAIL):
            do_pair((NCHUNK_FULL * L + j) * D, r_vec[j], c_vec[j], s_vec[j])

        pltpu.async_copy(obuf, out_hbm.at[pl.ds(fid * (P * D), P * D)],
                         osem).wait()


@jax.jit
def _ffm_sc(xt, tab3, w_flat, ridx, cidx):
    mesh = plsc.VectorSubcoreMesh(core_axis_name="c", subcore_axis_name="s")
    return pl.kernel(
        _body,
        out_type=jax.ShapeDtypeStruct((F * P * D,), jnp.float32),
        mesh=mesh,
        compiler_params=pltpu.CompilerParams(needs_layout_passes=False),
        scratch_types=[
            pltpu.VMEM((F * BPAD,), jnp.int32),
            pltpu.VMEM((F * K,), jnp.float32),
            pltpu.VMEM((PPAD,), jnp.int32),
            pltpu.VMEM((PPAD,), jnp.int32),
            pltpu.VMEM((B * D,), jnp.float32),
            pltpu.VMEM((P * D,), jnp.float32),
            [pltpu.VMEM((8, D), jnp.float32) for _ in range(NBUF)],
            [pltpu.SemaphoreType.DMA for _ in range(NBUF)],
            pltpu.SemaphoreType.DMA,
        ],
    )(xt, tab3, w_flat, ridx, cidx)


def kernel(x, tables, W):
    xt = jnp.pad(x.T.astype(jnp.int32), ((0, 0), (0, BPAD - B))).reshape(-1)
    tab3 = tables.reshape(F * V // 8, 8, D)
    w_flat = W.reshape(-1)
    r_np, c_np = np.triu_indices(B, k=1)
    ridx = jnp.pad(jnp.asarray(r_np, jnp.int32), (0, PPAD - P))
    cidx = jnp.pad(jnp.asarray(c_np, jnp.int32), (0, PPAD - P))
    return _ffm_sc(xt, tab3, w_flat, ridx, cidx).reshape(F, P * D)
